# Initial kernel scaffold; baseline (speedup 1.0000x reference)
import functools

import jax
import jax.numpy as jnp
from jax import lax
from jax.experimental import pallas as pl
from jax.experimental.pallas import tpu as pltpu
from jax.experimental.pallas import tpu_sc as plsc

_V = 50003
_ES = 32
_DL = 64
_E = 800000

_NC, _NS = 2, 16
_NW = _NC * _NS
_EPW = 25088
_EPAD = _NW * _EPW
_ER = _EPAD // 128
_RPW = _ER // _NW
_VPAD = 50176
_VPT = _VPAD // _NS

_CH = 512
_KR = _CH // 128
_NCHUNK = _EPW // _CH

_BE = 2048
_NEB = _EPAD // _BE
_BV = 3136
_NVB = _VPAD // _BV

_mesh = plsc.VectorSubcoreMesh(
    core_axis_name="c", subcore_axis_name="s", num_cores=_NC, num_subcores=_NS
)


def _wid():
    return lax.axis_index("s") * _NC + lax.axis_index("c")


def _make_gather(n_tab):
    scratch = []
    for _ in range(n_tab):
        scratch.append(pltpu.VMEM((_KR, 128), jnp.int32))
        scratch.append(pltpu.VMEM((_CH, _ES), jnp.float32))
    scratch.append(pltpu.SemaphoreType.DMA)
    out_type = [jax.ShapeDtypeStruct((_EPAD, _ES), jnp.float32)] * n_tab

    @functools.partial(pl.kernel, out_type=out_type, mesh=_mesh,
                       scratch_types=scratch)
    def k(*refs):
        tabs = refs[:n_tab]
        idxs = refs[n_tab:2 * n_tab]
        outs = refs[2 * n_tab:3 * n_tab]
        bufs = refs[3 * n_tab:3 * n_tab + 2 * n_tab]
        sem = refs[-1]
        w = _wid()

        def body(i, carry):
            base = w * _EPW + i * _CH
            rowb = w * _RPW + i * _KR
            for t in range(n_tab):
                pltpu.sync_copy(idxs[t].at[pl.ds(rowb, _KR)], bufs[2 * t])
            ds = []
            for t in range(n_tab):
                for j in range(_KR):
                    ds.append(pltpu.async_copy(
                        tabs[t].at[bufs[2 * t].at[j]],
                        bufs[2 * t + 1].at[pl.ds(j * 128, 128)], sem))
            for d in ds:
                d.wait()
            for t in range(n_tab):
                pltpu.sync_copy(bufs[2 * t + 1], outs[t].at[pl.ds(base, _CH)])
            return carry

        lax.fori_loop(0, _NCHUNK, body, 0)

    return k


_gather3 = _make_gather(3)
_gather2 = _make_gather(2)


@functools.partial(
    pl.kernel,
    out_type=jax.ShapeDtypeStruct((_NW, _VPAD), jnp.float32),
    mesh=_mesh,
    scratch_types=[
        pltpu.VMEM((_VPAD,), jnp.float32),
        pltpu.VMEM((_RPW, 128), jnp.int32),
        pltpu.VMEM((_RPW, 128), jnp.int32),
    ],
)
def _present(subj_hbm, obj_hbm, out_hbm, mask, sbuf, obuf):
    w = _wid()
    zeros16 = jnp.zeros((16,), jnp.float32)
    ones16 = jnp.ones((16,), jnp.float32)

    def zbody(i, c):
        off = pl.multiple_of(i * 128, 128)
        for u in range(8):
            mask[pl.ds(off + u * 16, 16)] = zeros16
        return c

    lax.fori_loop(0, _VPAD // 128, zbody, 0)

    pltpu.sync_copy(subj_hbm.at[pl.ds(w * _RPW, _RPW)], sbuf)
    pltpu.sync_copy(obj_hbm.at[pl.ds(w * _RPW, _RPW)], obuf)

    def sbody(r, c):
        for u in range(8):
            iv = sbuf[r, pl.ds(u * 16, 16)]
            plsc.store_scatter(mask, [iv], ones16)
            jv = obuf[r, pl.ds(u * 16, 16)]
            plsc.store_scatter(mask, [jv], ones16)
        return c

    lax.fori_loop(0, _RPW, sbody, 0)
    pltpu.sync_copy(mask, out_hbm.at[w])


@functools.partial(
    pl.kernel,
    out_type=jax.ShapeDtypeStruct((_NC, _VPAD, _ES), jnp.float32),
    mesh=_mesh,
    scratch_types=[
        pltpu.VMEM((_KR, 128), jnp.int32),
        pltpu.VMEM((_CH, _ES), jnp.float32),
        pltpu.VMEM((196, _ES), jnp.float32),
        pltpu.VMEM_SHARED((_VPAD, _ES), jnp.float32),
    ],
)
def _scatter(msg_hbm, obj_hbm, out_hbm, idxb, msgb, zb, shared):
    c = lax.axis_index("c")
    s = lax.axis_index("s")
    w = _wid()
    zeros16 = jnp.zeros((16,), jnp.float32)

    def zvbody(i, cr):
        zb[i, pl.ds(0, 16)] = zeros16
        zb[i, pl.ds(16, 16)] = zeros16
        return cr

    lax.fori_loop(0, 196, zvbody, 0)

    def zsbody(i, cr):
        pltpu.sync_copy(zb, shared.at[pl.ds(s * _VPT + i * 196, 196)])
        return cr

    lax.fori_loop(0, _VPT // 196, zsbody, 0)
    plsc.subcore_barrier()

    def body(i, cr):
        base = w * _EPW + i * _CH
        rowb = w * _RPW + i * _KR
        pltpu.sync_copy(obj_hbm.at[pl.ds(rowb, _KR)], idxb)
        pltpu.sync_copy(msg_hbm.at[pl.ds(base, _CH)], msgb)
        for j in range(_KR):
            pltpu.sync_copy(msgb.at[pl.ds(j * 128, 128)],
                            shared.at[idxb.at[j]], add=True)
        return cr

    lax.fori_loop(0, _NCHUNK, body, 0)
    plsc.subcore_barrier()

    def fbody(i, cr):
        off = s * _VPT + i * 196
        pltpu.sync_copy(shared.at[pl.ds(off, 196)],
                        out_hbm.at[c, pl.ds(off, 196)])
        return cr

    lax.fori_loop(0, _VPT // 196, fbody, 0)


def _gelu(x):
    return x * 0.5 * (1.0 + lax.erf(x * 0.7071067811865476))


def _mlp1_body(gs, ge, go, w1a, w1b, w1c, b1, w2, b2, out):
    pre = (jnp.dot(gs[...], w1a[...], preferred_element_type=jnp.float32)
           + jnp.dot(ge[...], w1b[...], preferred_element_type=jnp.float32)
           + jnp.dot(go[...], w1c[...], preferred_element_type=jnp.float32)
           + b1[...])
    h = _gelu(pre)
    out[...] = jnp.dot(h, w2[...], preferred_element_type=jnp.float32) + b2[...]


def _mlp1(gs, ge, go, w1a, w1b, w1c, b1, w2, b2):
    eb = pl.BlockSpec((_BE, _ES), lambda i: (i, 0))
    full = lambda shape: pl.BlockSpec(shape, lambda i: tuple(0 for _ in shape))
    return pl.pallas_call(
        _mlp1_body,
        grid=(_NEB,),
        in_specs=[eb, eb, eb, full((_ES, 64)), full((_ES, 64)), full((_ES, 64)),
                  full((1, 64)), full((64, _ES)), full((1, _ES))],
        out_specs=eb,
        out_shape=jax.ShapeDtypeStruct((_EPAD, _ES), jnp.float32),
    )(gs, ge, go, w1a, w1b, w1c, b1, w2, b2)


def _mlp2_body(gs, ge, go, w1a, w1b, w1c, b1, out):
    i = pl.program_id(0)
    pre = (jnp.dot(gs[...], w1a[...], preferred_element_type=jnp.float32)
           + jnp.dot(ge[...], w1b[...], preferred_element_type=jnp.float32)
           + jnp.dot(go[...], w1c[...], preferred_element_type=jnp.float32)
           + b1[...])
    h = _gelu(pre)
    row = i * _BE + lax.broadcasted_iota(jnp.int32, (_BE, 1), 0)
    h = jnp.where(row < _E, h, 0.0)
    part = jnp.sum(h, axis=0, keepdims=True)

    @pl.when(i == 0)
    def _():
        out[...] = jnp.zeros_like(out)

    out[...] += part


def _mlp2(gs, ge, go, w1a, w1b, w1c, b1):
    eb = pl.BlockSpec((_BE, _ES), lambda i: (i, 0))
    full = lambda shape: pl.BlockSpec(shape, lambda i: tuple(0 for _ in shape))
    return pl.pallas_call(
        _mlp2_body,
        grid=(_NEB,),
        in_specs=[eb, eb, eb, full((_ES, 64)), full((_ES, 64)), full((_ES, 64)),
                  full((1, 64))],
        out_specs=pl.BlockSpec((1, 64), lambda i: (0, 0)),
        out_shape=jax.ShapeDtypeStruct((1, 64), jnp.float32),
    )(gs, ge, go, w1a, w1b, w1c, b1)


def _prep2_body(sym, d0, d1, out):
    i = pl.program_id(0)
    row = i * _BV + lax.broadcasted_iota(jnp.int32, (_BV, 1), 0)
    out[...] = jnp.where(row < _V, sym[...] + d0[...] + d1[...], 0.0)


def _prep2(sym, d0, d1):
    vb = pl.BlockSpec((_BV, _ES), lambda i: (i, 0))
    return pl.pallas_call(
        _prep2_body,
        grid=(_NVB,),
        in_specs=[vb, vb, vb],
        out_specs=vb,
        out_shape=jax.ShapeDtypeStruct((_VPAD, _ES), jnp.float32),
    )(sym, d0, d1)


def _final_body(pmask, sym, d0, d1, sumh2, w21, b21, wl, bl, out,
                acc, cnt):
    i = pl.program_id(0)
    row = i * _BV + lax.broadcasted_iota(jnp.int32, (1, _BV), 1)
    rowmask = (row < _V).astype(jnp.float32)
    seen = jnp.max(pmask[...], axis=0, keepdims=True)
    pm = jnp.where(seen > 0.0, 1.0, 0.0) * rowmask
    part = (jnp.dot(pm, sym[...], preferred_element_type=jnp.float32)
            + jnp.dot(rowmask, d0[...] + d1[...],
                      preferred_element_type=jnp.float32))

    @pl.when(i == 0)
    def _():
        acc[...] = jnp.zeros_like(acc)
        cnt[0, 0] = 0.0

    acc[...] += part
    cnt[0, 0] += jnp.sum(pm)

    @pl.when(i == _NVB - 1)
    def _():
        msg2 = (jnp.dot(sumh2[...], w21[...],
                        preferred_element_type=jnp.float32)
                + float(_E) * b21[...])
        mean = (acc[...] + msg2) / cnt[0, 0]
        out[...] = (jnp.dot(mean, wl[...],
                            preferred_element_type=jnp.float32) + bl[...])


def _final(pmask, sym, d0, d1, sumh2, w21, b21, wl, bl):
    vb = pl.BlockSpec((_BV, _ES), lambda i: (i, 0))
    full = lambda shape: pl.BlockSpec(shape, lambda i: tuple(0 for _ in shape))
    return pl.pallas_call(
        _final_body,
        grid=(_NVB,),
        in_specs=[pl.BlockSpec((_NW, _BV), lambda i: (0, i)), vb, vb, vb,
                  full((1, 64)), full((64, _ES)), full((1, _ES)),
                  full((_ES, _DL)), full((1, _DL))],
        out_specs=pl.BlockSpec((1, _DL), lambda i: (0, 0)),
        out_shape=jax.ShapeDtypeStruct((1, _DL), jnp.float32),
        scratch_shapes=[pltpu.VMEM((1, _ES), jnp.float32),
                        pltpu.VMEM((1, 1), jnp.float32)],
    )(pmask, sym, d0, d1, sumh2, w21, b21, wl, bl)


def kernel(facts, sym_emb, edge_emb, W1_0, b1_0, W2_0, b2_0,
           W1_1, b1_1, W2_1, b2_1, Wl, bl):
    pad_e = _EPAD - _E
    subj = jnp.concatenate(
        [facts[:, 0], jnp.full((pad_e,), _V, jnp.int32)]).reshape(_ER, 128)
    pred = jnp.concatenate(
        [facts[:, 1], jnp.full((pad_e,), _V, jnp.int32)]).reshape(_ER, 128)
    obj = jnp.concatenate(
        [facts[:, 2], jnp.full((pad_e,), _V, jnp.int32)]).reshape(_ER, 128)

    zpad = jnp.zeros((_VPAD - _V, _ES), jnp.float32)
    sym_p = jnp.concatenate([sym_emb, zpad], axis=0)
    edge_p = jnp.concatenate([edge_emb, zpad], axis=0)

    b1_0r = b1_0.reshape(1, 64)
    b2_0r = b2_0.reshape(1, _ES)
    b1_1r = b1_1.reshape(1, 64)
    b2_1r = b2_1.reshape(1, _ES)
    blr = bl.reshape(1, _DL)

    gs1, ge, go1 = _gather3(sym_p, edge_p, sym_p, subj, pred, obj)
    pmask = _present(subj, obj)
    msg1 = _mlp1(gs1, ge, go1, W1_0[:_ES], W1_0[_ES:2 * _ES], W1_0[2 * _ES:],
                 b1_0r, W2_0, b2_0r)
    delta = _scatter(msg1, obj)
    nodes2 = _prep2(sym_p, delta[0], delta[1])
    gs2, go2 = _gather2(nodes2, nodes2, subj, obj)
    sumh2 = _mlp2(gs2, ge, go2, W1_1[:_ES], W1_1[_ES:2 * _ES], W1_1[2 * _ES:],
                  b1_1r)
    return _final(pmask, sym_p, delta[0], delta[1], sumh2, W2_1, b2_1r,
                  Wl, blr)


# speedup vs baseline: 5.8732x; 5.8732x over previous
import functools

import jax
import jax.numpy as jnp
from jax import lax
from jax.experimental import pallas as pl
from jax.experimental.pallas import tpu as pltpu
from jax.experimental.pallas import tpu_sc as plsc

_V = 50003
_ES = 32
_DL = 64
_E = 800000

_NC, _NS = 2, 16
_NW = _NC * _NS
_EPW = 25088
_EPAD = _NW * _EPW
_ER = _EPAD // 128
_RPW = _ER // _NW
_VPAD = 50176
_VPT = _VPAD // _NS

_CH = 512
_KR = _CH // 128
_NCHUNK = _EPW // _CH

_BE = 2048
_NEB = _EPAD // _BE
_BV = 6272
_NVB = _VPAD // _BV

_mesh = plsc.VectorSubcoreMesh(
    core_axis_name="c", subcore_axis_name="s", num_cores=_NC, num_subcores=_NS
)
_sc_params = pltpu.CompilerParams(use_tc_tiling_on_sc=False,
                                  needs_layout_passes=False)


def _wid():
    return lax.axis_index("s") * _NC + lax.axis_index("c")


def _make_gather(n_tab):
    scratch = []
    for _ in range(n_tab):
        scratch.append(pltpu.VMEM((_KR, 128), jnp.int32))
        scratch.append(pltpu.VMEM((_CH, _ES), jnp.float32))
    scratch.append(pltpu.SemaphoreType.DMA)
    out_type = [jax.ShapeDtypeStruct((_EPAD, _ES), jnp.float32)] * n_tab

    @functools.partial(pl.kernel, out_type=out_type, mesh=_mesh,
                       scratch_types=scratch, compiler_params=_sc_params)
    def k(*refs):
        tabs = refs[:n_tab]
        idxs = refs[n_tab:2 * n_tab]
        outs = refs[2 * n_tab:3 * n_tab]
        bufs = refs[3 * n_tab:3 * n_tab + 2 * n_tab]
        sem = refs[-1]
        w = _wid()

        def body(i, carry):
            base = w * _EPW + i * _CH
            rowb = w * _RPW + i * _KR
            for t in range(n_tab):
                pltpu.sync_copy(idxs[t].at[pl.ds(rowb, _KR)], bufs[2 * t])
            ds = []
            for t in range(n_tab):
                for j in range(_KR):
                    ds.append(pltpu.async_copy(
                        tabs[t].at[bufs[2 * t].at[j]],
                        bufs[2 * t + 1].at[pl.ds(j * 128, 128)], sem))
            for d in ds:
                d.wait()
            for t in range(n_tab):
                pltpu.sync_copy(bufs[2 * t + 1], outs[t].at[pl.ds(base, _CH)])
            return carry

        lax.fori_loop(0, _NCHUNK, body, 0)

    return k


_gather3 = _make_gather(3)
_gather2 = _make_gather(2)


@functools.partial(
    pl.kernel,
    out_type=jax.ShapeDtypeStruct((_NW, _VPAD), jnp.float32),
    mesh=_mesh,
    compiler_params=_sc_params,
    scratch_types=[
        pltpu.VMEM((_VPAD,), jnp.float32),
        pltpu.VMEM((_RPW, 128), jnp.int32),
        pltpu.VMEM((_RPW, 128), jnp.int32),
    ],
)
def _present(subj_hbm, obj_hbm, out_hbm, mask, sbuf, obuf):
    w = _wid()
    zeros16 = jnp.zeros((16,), jnp.float32)
    ones16 = jnp.ones((16,), jnp.float32)

    def zbody(i, c):
        off = pl.multiple_of(i * 128, 128)
        for u in range(8):
            mask[pl.ds(off + u * 16, 16)] = zeros16
        return c

    lax.fori_loop(0, _VPAD // 128, zbody, 0)

    pltpu.sync_copy(subj_hbm.at[pl.ds(w * _RPW, _RPW)], sbuf)
    pltpu.sync_copy(obj_hbm.at[pl.ds(w * _RPW, _RPW)], obuf)

    def sbody(r, c):
        for u in range(8):
            iv = sbuf[r, pl.ds(u * 16, 16)]
            plsc.store_scatter(mask, [iv], ones16)
            jv = obuf[r, pl.ds(u * 16, 16)]
            plsc.store_scatter(mask, [jv], ones16)
        return c

    lax.fori_loop(0, _RPW, sbody, 0)
    pltpu.sync_copy(mask, out_hbm.at[w])


@functools.partial(
    pl.kernel,
    out_type=jax.ShapeDtypeStruct((_NC, _VPAD, _ES), jnp.float32),
    mesh=_mesh,
    compiler_params=_sc_params,
    scratch_types=[
        pltpu.VMEM((_KR, 128), jnp.int32),
        pltpu.VMEM((_CH, _ES), jnp.float32),
        pltpu.VMEM((196, _ES), jnp.float32),
        pltpu.VMEM_SHARED((_VPAD, _ES), jnp.float32),
    ],
)
def _scatter(msg_hbm, obj_hbm, out_hbm, idxb, msgb, zb, shared):
    c = lax.axis_index("c")
    s = lax.axis_index("s")
    w = _wid()
    zeros16 = jnp.zeros((16,), jnp.float32)

    def zvbody(i, cr):
        zb[i, pl.ds(0, 16)] = zeros16
        zb[i, pl.ds(16, 16)] = zeros16
        return cr

    lax.fori_loop(0, 196, zvbody, 0)

    def zsbody(i, cr):
        pltpu.sync_copy(zb, shared.at[pl.ds(s * _VPT + i * 196, 196)])
        return cr

    lax.fori_loop(0, _VPT // 196, zsbody, 0)
    plsc.subcore_barrier()

    def body(i, cr):
        base = w * _EPW + i * _CH
        rowb = w * _RPW + i * _KR
        pltpu.sync_copy(obj_hbm.at[pl.ds(rowb, _KR)], idxb)
        pltpu.sync_copy(msg_hbm.at[pl.ds(base, _CH)], msgb)
        for j in range(_KR):
            pltpu.sync_copy(msgb.at[pl.ds(j * 128, 128)],
                            shared.at[idxb.at[j]], add=True)
        return cr

    lax.fori_loop(0, _NCHUNK, body, 0)
    plsc.subcore_barrier()

    def fbody(i, cr):
        off = s * _VPT + i * 196
        pltpu.sync_copy(shared.at[pl.ds(off, 196)],
                        out_hbm.at[c, pl.ds(off, 196)])
        return cr

    lax.fori_loop(0, _VPT // 196, fbody, 0)


def _gelu(x):
    return x * 0.5 * (1.0 + lax.erf(x * 0.7071067811865476))


def _mlp1_body(gs, ge, go, w1a, w1b, w1c, b1, w2, b2, out):
    pre = (jnp.dot(gs[...], w1a[...], preferred_element_type=jnp.float32)
           + jnp.dot(ge[...], w1b[...], preferred_element_type=jnp.float32)
           + jnp.dot(go[...], w1c[...], preferred_element_type=jnp.float32)
           + b1[...])
    h = _gelu(pre)
    out[...] = jnp.dot(h, w2[...], preferred_element_type=jnp.float32) + b2[...]


def _mlp1(gs, ge, go, w1a, w1b, w1c, b1, w2, b2):
    eb = pl.BlockSpec((_BE, _ES), lambda i: (i, 0))
    full = lambda shape: pl.BlockSpec(shape, lambda i: tuple(0 for _ in shape))
    return pl.pallas_call(
        _mlp1_body,
        grid=(_NEB,),
        in_specs=[eb, eb, eb, full((_ES, 64)), full((_ES, 64)), full((_ES, 64)),
                  full((1, 64)), full((64, _ES)), full((1, _ES))],
        out_specs=eb,
        out_shape=jax.ShapeDtypeStruct((_EPAD, _ES), jnp.float32),
    )(gs, ge, go, w1a, w1b, w1c, b1, w2, b2)


def _mlp2_body(gs, ge, go, w1a, w1b, w1c, b1, out):
    i = pl.program_id(0)
    pre = (jnp.dot(gs[...], w1a[...], preferred_element_type=jnp.float32)
           + jnp.dot(ge[...], w1b[...], preferred_element_type=jnp.float32)
           + jnp.dot(go[...], w1c[...], preferred_element_type=jnp.float32)
           + b1[...])
    h = _gelu(pre)
    row = i * _BE + lax.broadcasted_iota(jnp.int32, (_BE, 1), 0)
    h = jnp.where(row < _E, h, 0.0)
    part = jnp.sum(h, axis=0, keepdims=True)

    @pl.when(i == 0)
    def _():
        out[...] = jnp.zeros_like(out)

    out[...] += part


def _mlp2(gs, ge, go, w1a, w1b, w1c, b1):
    eb = pl.BlockSpec((_BE, _ES), lambda i: (i, 0))
    full = lambda shape: pl.BlockSpec(shape, lambda i: tuple(0 for _ in shape))
    return pl.pallas_call(
        _mlp2_body,
        grid=(_NEB,),
        in_specs=[eb, eb, eb, full((_ES, 64)), full((_ES, 64)), full((_ES, 64)),
                  full((1, 64))],
        out_specs=pl.BlockSpec((1, 64), lambda i: (0, 0)),
        out_shape=jax.ShapeDtypeStruct((1, 64), jnp.float32),
    )(gs, ge, go, w1a, w1b, w1c, b1)


def _prep2_body(sym, d0, d1, out):
    i = pl.program_id(0)
    row = i * _BV + lax.broadcasted_iota(jnp.int32, (_BV, 1), 0)
    out[...] = jnp.where(row < _V, sym[...] + d0[...] + d1[...], 0.0)


def _prep2(sym, d0, d1):
    vb = pl.BlockSpec((_BV, _ES), lambda i: (i, 0))
    return pl.pallas_call(
        _prep2_body,
        grid=(_NVB,),
        in_specs=[vb, vb, vb],
        out_specs=vb,
        out_shape=jax.ShapeDtypeStruct((_VPAD, _ES), jnp.float32),
    )(sym, d0, d1)


def _final_body(pmask, sym, d0, d1, sumh2, w21, b21, wl, bl, out,
                acc, cnt):
    i = pl.program_id(0)
    row = i * _BV + lax.broadcasted_iota(jnp.int32, (1, _BV), 1)
    rowmask = (row < _V).astype(jnp.float32)
    seen = jnp.max(pmask[...], axis=0, keepdims=True)
    pm = jnp.where(seen > 0.0, 1.0, 0.0) * rowmask
    part = (jnp.dot(pm, sym[...], preferred_element_type=jnp.float32)
            + jnp.dot(rowmask, d0[...] + d1[...],
                      preferred_element_type=jnp.float32))

    @pl.when(i == 0)
    def _():
        acc[...] = jnp.zeros_like(acc)
        cnt[...] = jnp.zeros_like(cnt)

    acc[...] += part
    cnt[...] += jnp.sum(pm).reshape(1, 1)

    @pl.when(i == _NVB - 1)
    def _():
        msg2 = (jnp.dot(sumh2[...], w21[...],
                        preferred_element_type=jnp.float32)
                + float(_E) * b21[...])
        mean = (acc[...] + msg2) / cnt[...]
        out[...] = (jnp.dot(mean, wl[...],
                            preferred_element_type=jnp.float32) + bl[...])


def _final(pmask, sym, d0, d1, sumh2, w21, b21, wl, bl):
    vb = pl.BlockSpec((_BV, _ES), lambda i: (i, 0))
    full = lambda shape: pl.BlockSpec(shape, lambda i: tuple(0 for _ in shape))
    return pl.pallas_call(
        _final_body,
        grid=(_NVB,),
        in_specs=[pl.BlockSpec((_NW, _BV), lambda i: (0, i)), vb, vb, vb,
                  full((1, 64)), full((64, _ES)), full((1, _ES)),
                  full((_ES, _DL)), full((1, _DL))],
        out_specs=pl.BlockSpec((1, _DL), lambda i: (0, 0)),
        out_shape=jax.ShapeDtypeStruct((1, _DL), jnp.float32),
        scratch_shapes=[pltpu.VMEM((1, _ES), jnp.float32),
                        pltpu.VMEM((1, 1), jnp.float32)],
    )(pmask, sym, d0, d1, sumh2, w21, b21, wl, bl)


def kernel(facts, sym_emb, edge_emb, W1_0, b1_0, W2_0, b2_0,
           W1_1, b1_1, W2_1, b2_1, Wl, bl):
    pad_e = _EPAD - _E
    subj = jnp.concatenate(
        [facts[:, 0], jnp.full((pad_e,), _V, jnp.int32)]).reshape(_ER, 128)
    pred = jnp.concatenate(
        [facts[:, 1], jnp.full((pad_e,), _V, jnp.int32)]).reshape(_ER, 128)
    obj = jnp.concatenate(
        [facts[:, 2], jnp.full((pad_e,), _V, jnp.int32)]).reshape(_ER, 128)

    zpad = jnp.zeros((_VPAD - _V, _ES), jnp.float32)
    sym_p = jnp.concatenate([sym_emb, zpad], axis=0)
    edge_p = jnp.concatenate([edge_emb, zpad], axis=0)

    b1_0r = b1_0.reshape(1, 64)
    b2_0r = b2_0.reshape(1, _ES)
    b1_1r = b1_1.reshape(1, 64)
    b2_1r = b2_1.reshape(1, _ES)
    blr = bl.reshape(1, _DL)

    gs1, ge, go1 = _gather3(sym_p, edge_p, sym_p, subj, pred, obj)
    pmask = _present(subj, obj)
    msg1 = _mlp1(gs1, ge, go1, W1_0[:_ES], W1_0[_ES:2 * _ES], W1_0[2 * _ES:],
                 b1_0r, W2_0, b2_0r)
    delta = _scatter(msg1, obj)
    nodes2 = _prep2(sym_p, delta[0], delta[1])
    gs2, go2 = _gather2(nodes2, nodes2, subj, obj)
    sumh2 = _mlp2(gs2, ge, go2, W1_1[:_ES], W1_1[_ES:2 * _ES], W1_1[2 * _ES:],
                  b1_1r)
    return _final(pmask, sym_p, delta[0], delta[1], sumh2, W2_1, b2_1r,
                  Wl, blr)


# speedup vs baseline: 6.1622x; 1.0492x over previous
import functools

import jax
import jax.numpy as jnp
from jax import lax
from jax.experimental import pallas as pl
from jax.experimental.pallas import tpu as pltpu
from jax.experimental.pallas import tpu_sc as plsc

_V = 50003
_ES = 32
_DL = 64
_E = 800000

_NC, _NS = 2, 16
_NW = _NC * _NS
_EPW = 25088
_EPAD = _NW * _EPW
_ER = _EPAD // 128
_RPW = _ER // _NW
_VPAD = 50176
_VPT = _VPAD // _NS

_BE = 2048
_NEB = _EPAD // _BE
_BV = 6272
_NVB = _VPAD // _BV

_mesh = plsc.VectorSubcoreMesh(
    core_axis_name="c", subcore_axis_name="s", num_cores=_NC, num_subcores=_NS
)
_sc_params = pltpu.CompilerParams(use_tc_tiling_on_sc=False,
                                  needs_layout_passes=False)


def _wid():
    return lax.axis_index("s") * _NC + lax.axis_index("c")


def _make_gather(n_tab, ch):
    kr = ch // 128
    nch = _EPW // ch
    scratch = []
    for _ in range(n_tab):
        scratch.append(pltpu.VMEM((_RPW, 128), jnp.int32))
    for _p in range(2):
        for _ in range(n_tab):
            scratch.append(pltpu.VMEM((ch, _ES), jnp.float32))
    scratch.append(pltpu.SemaphoreType.DMA)
    scratch.append(pltpu.SemaphoreType.DMA)
    out_type = [jax.ShapeDtypeStruct((_EPAD, _ES), jnp.float32)] * n_tab

    @functools.partial(pl.kernel, out_type=out_type, mesh=_mesh,
                       scratch_types=scratch, compiler_params=_sc_params)
    def k(*refs):
        tabs = refs[:n_tab]
        idxs = refs[n_tab:2 * n_tab]
        outs = refs[2 * n_tab:3 * n_tab]
        ib = refs[3 * n_tab:4 * n_tab]
        rb = (refs[4 * n_tab:5 * n_tab], refs[5 * n_tab:6 * n_tab])
        sems = refs[6 * n_tab:6 * n_tab + 2]
        w = _wid()

        for t in range(n_tab):
            pltpu.sync_copy(idxs[t].at[pl.ds(w * _RPW, _RPW)], ib[t])

        def stream(c, p, start):
            for t in range(n_tab):
                for j in range(kr):
                    d = pltpu.make_async_copy(
                        tabs[t].at[ib[t].at[c * kr + j]],
                        rb[p][t].at[pl.ds(j * 128, 128)], sems[p])
                    if start:
                        d.start()
                    else:
                        d.wait()

        def out(c, p):
            base = w * _EPW + c * ch
            for t in range(n_tab):
                pltpu.sync_copy(rb[p][t], outs[t].at[pl.ds(base, ch)])

        stream(0, 0, True)

        def body(j, cr):
            ca = 2 * j + 1
            stream(ca, 1, True)
            stream(ca - 1, 0, False)
            out(ca - 1, 0)
            cb = 2 * j + 2
            stream(cb, 0, True)
            stream(cb - 1, 1, False)
            out(cb - 1, 1)
            return cr

        lax.fori_loop(0, (nch - 1) // 2, body, 0)
        cl = nch - 1
        if nch % 2 == 0:
            stream(cl, 1, True)
            stream(cl - 1, 0, False)
            out(cl - 1, 0)
            stream(cl, 1, False)
            out(cl, 1)
        else:
            stream(cl, 0, False)
            out(cl, 0)

    return k


_gather3 = _make_gather(3, 256)
_gather2 = _make_gather(2, 512)


@functools.partial(
    pl.kernel,
    out_type=jax.ShapeDtypeStruct((_NW, _VPAD), jnp.float32),
    mesh=_mesh,
    compiler_params=_sc_params,
    scratch_types=[
        pltpu.VMEM((_VPAD,), jnp.float32),
        pltpu.VMEM((_RPW, 128), jnp.int32),
        pltpu.VMEM((_RPW, 128), jnp.int32),
    ],
)
def _present(subj_hbm, obj_hbm, out_hbm, mask, sbuf, obuf):
    w = _wid()
    zeros16 = jnp.zeros((16,), jnp.float32)
    ones16 = jnp.ones((16,), jnp.float32)

    def zbody(i, c):
        off = pl.multiple_of(i * 128, 128)
        for u in range(8):
            mask[pl.ds(off + u * 16, 16)] = zeros16
        return c

    lax.fori_loop(0, _VPAD // 128, zbody, 0)

    pltpu.sync_copy(subj_hbm.at[pl.ds(w * _RPW, _RPW)], sbuf)
    pltpu.sync_copy(obj_hbm.at[pl.ds(w * _RPW, _RPW)], obuf)

    def sbody(r, c):
        for u in range(8):
            iv = sbuf[r, pl.ds(u * 16, 16)]
            plsc.store_scatter(mask, [iv], ones16)
            jv = obuf[r, pl.ds(u * 16, 16)]
            plsc.store_scatter(mask, [jv], ones16)
        return c

    lax.fori_loop(0, _RPW, sbody, 0)
    pltpu.sync_copy(mask, out_hbm.at[w])


_SCH = 256
_SKR = _SCH // 128
_SNCH = _EPW // _SCH


@functools.partial(
    pl.kernel,
    out_type=jax.ShapeDtypeStruct((_NC, _VPAD, _ES), jnp.float32),
    mesh=_mesh,
    compiler_params=_sc_params,
    scratch_types=[
        pltpu.VMEM((_SKR, 128), jnp.int32),
        pltpu.VMEM((_SKR, 128), jnp.int32),
        pltpu.VMEM((_SCH, _ES), jnp.float32),
        pltpu.VMEM((_SCH, _ES), jnp.float32),
        pltpu.VMEM((196, _ES), jnp.float32),
        pltpu.VMEM_SHARED((_VPAD, _ES), jnp.float32),
        pltpu.SemaphoreType.DMA,
        pltpu.SemaphoreType.DMA,
    ],
)
def _scatter(msg_hbm, obj_hbm, out_hbm, ib0, ib1, mb0, mb1, zb, shared,
             sem0, sem1):
    c = lax.axis_index("c")
    s = lax.axis_index("s")
    w = _wid()
    ib = (ib0, ib1)
    mb = (mb0, mb1)
    sems = (sem0, sem1)
    zeros16 = jnp.zeros((16,), jnp.float32)

    def zvbody(i, cr):
        zb[i, pl.ds(0, 16)] = zeros16
        zb[i, pl.ds(16, 16)] = zeros16
        return cr

    lax.fori_loop(0, 196, zvbody, 0)

    def zsbody(i, cr):
        pltpu.sync_copy(zb, shared.at[pl.ds(s * _VPT + i * 196, 196)])
        return cr

    lax.fori_loop(0, _VPT // 196, zsbody, 0)
    plsc.subcore_barrier()

    def load(ci, p):
        pltpu.sync_copy(obj_hbm.at[pl.ds(w * _RPW + ci * _SKR, _SKR)], ib[p])
        pltpu.sync_copy(msg_hbm.at[pl.ds(w * _EPW + ci * _SCH, _SCH)], mb[p])

    def fire(ci, p):
        for j in range(_SKR):
            pltpu.async_copy(mb[p].at[pl.ds(j * 128, 128)],
                             shared.at[ib[p].at[j]], sems[p], add=True)

    def drain(p):
        for j in range(_SKR):
            pltpu.make_async_copy(mb[p].at[pl.ds(j * 128, 128)],
                                  shared.at[ib[p].at[j]], sems[p]).wait()

    load(0, 0)
    fire(0, 0)
    load(1, 1)
    fire(1, 1)
    drain(0)
    load(2, 0)
    fire(2, 0)

    def body(j, cr):
        ca = 2 * j + 3
        drain(1)
        load(ca, 1)
        fire(ca, 1)
        cb = 2 * j + 4
        drain(0)
        load(cb, 0)
        fire(cb, 0)
        return cr

    lax.fori_loop(0, (_SNCH - 4) // 2, body, 0)
    cl = _SNCH - 1
    drain(1)
    load(cl, 1)
    fire(cl, 1)
    drain(0)
    drain(1)
    plsc.subcore_barrier()

    def fbody(i, cr):
        off = s * _VPT + i * 196
        pltpu.sync_copy(shared.at[pl.ds(off, 196)],
                        out_hbm.at[c, pl.ds(off, 196)])
        return cr

    lax.fori_loop(0, _VPT // 196, fbody, 0)


def _gelu(x):
    return x * 0.5 * (1.0 + lax.erf(x * 0.7071067811865476))


def _mlp1_body(gs, ge, go, w1a, w1b, w1c, b1, w2, b2, out):
    pre = (jnp.dot(gs[...], w1a[...], preferred_element_type=jnp.float32)
           + jnp.dot(ge[...], w1b[...], preferred_element_type=jnp.float32)
           + jnp.dot(go[...], w1c[...], preferred_element_type=jnp.float32)
           + b1[...])
    h = _gelu(pre)
    out[...] = jnp.dot(h, w2[...], preferred_element_type=jnp.float32) + b2[...]


def _mlp1(gs, ge, go, w1a, w1b, w1c, b1, w2, b2):
    eb = pl.BlockSpec((_BE, _ES), lambda i: (i, 0))
    full = lambda shape: pl.BlockSpec(shape, lambda i: tuple(0 for _ in shape))
    return pl.pallas_call(
        _mlp1_body,
        grid=(_NEB,),
        in_specs=[eb, eb, eb, full((_ES, 64)), full((_ES, 64)), full((_ES, 64)),
                  full((1, 64)), full((64, _ES)), full((1, _ES))],
        out_specs=eb,
        out_shape=jax.ShapeDtypeStruct((_EPAD, _ES), jnp.float32),
    )(gs, ge, go, w1a, w1b, w1c, b1, w2, b2)


def _mlp2_body(gs, ge, go, w1a, w1b, w1c, b1, out):
    i = pl.program_id(0)
    pre = (jnp.dot(gs[...], w1a[...], preferred_element_type=jnp.float32)
           + jnp.dot(ge[...], w1b[...], preferred_element_type=jnp.float32)
           + jnp.dot(go[...], w1c[...], preferred_element_type=jnp.float32)
           + b1[...])
    h = _gelu(pre)
    row = i * _BE + lax.broadcasted_iota(jnp.int32, (_BE, 1), 0)
    h = jnp.where(row < _E, h, 0.0)
    part = jnp.sum(h, axis=0, keepdims=True)

    @pl.when(i == 0)
    def _():
        out[...] = jnp.zeros_like(out)

    out[...] += part


def _mlp2(gs, ge, go, w1a, w1b, w1c, b1):
    eb = pl.BlockSpec((_BE, _ES), lambda i: (i, 0))
    full = lambda shape: pl.BlockSpec(shape, lambda i: tuple(0 for _ in shape))
    return pl.pallas_call(
        _mlp2_body,
        grid=(_NEB,),
        in_specs=[eb, eb, eb, full((_ES, 64)), full((_ES, 64)), full((_ES, 64)),
                  full((1, 64))],
        out_specs=pl.BlockSpec((1, 64), lambda i: (0, 0)),
        out_shape=jax.ShapeDtypeStruct((1, 64), jnp.float32),
    )(gs, ge, go, w1a, w1b, w1c, b1)


def _prep2_body(sym, d0, d1, out):
    i = pl.program_id(0)
    row = i * _BV + lax.broadcasted_iota(jnp.int32, (_BV, 1), 0)
    out[...] = jnp.where(row < _V, sym[...] + d0[...] + d1[...], 0.0)


def _prep2(sym, d0, d1):
    vb = pl.BlockSpec((_BV, _ES), lambda i: (i, 0))
    return pl.pallas_call(
        _prep2_body,
        grid=(_NVB,),
        in_specs=[vb, vb, vb],
        out_specs=vb,
        out_shape=jax.ShapeDtypeStruct((_VPAD, _ES), jnp.float32),
    )(sym, d0, d1)


def _final_body(pmask, sym, d0, d1, sumh2, w21, b21, wl, bl, out,
                acc, cnt):
    i = pl.program_id(0)
    row = i * _BV + lax.broadcasted_iota(jnp.int32, (1, _BV), 1)
    rowmask = (row < _V).astype(jnp.float32)
    seen = jnp.max(pmask[...], axis=0, keepdims=True)
    pm = jnp.where(seen > 0.0, 1.0, 0.0) * rowmask
    part = (jnp.dot(pm, sym[...], preferred_element_type=jnp.float32)
            + jnp.dot(rowmask, d0[...] + d1[...],
                      preferred_element_type=jnp.float32))

    @pl.when(i == 0)
    def _():
        acc[...] = jnp.zeros_like(acc)
        cnt[...] = jnp.zeros_like(cnt)

    acc[...] += part
    cnt[...] += jnp.sum(pm).reshape(1, 1)

    @pl.when(i == _NVB - 1)
    def _():
        msg2 = (jnp.dot(sumh2[...], w21[...],
                        preferred_element_type=jnp.float32)
                + float(_E) * b21[...])
        mean = (acc[...] + msg2) / cnt[...]
        out[...] = (jnp.dot(mean, wl[...],
                            preferred_element_type=jnp.float32) + bl[...])


def _final(pmask, sym, d0, d1, sumh2, w21, b21, wl, bl):
    vb = pl.BlockSpec((_BV, _ES), lambda i: (i, 0))
    full = lambda shape: pl.BlockSpec(shape, lambda i: tuple(0 for _ in shape))
    return pl.pallas_call(
        _final_body,
        grid=(_NVB,),
        in_specs=[pl.BlockSpec((_NW, _BV), lambda i: (0, i)), vb, vb, vb,
                  full((1, 64)), full((64, _ES)), full((1, _ES)),
                  full((_ES, _DL)), full((1, _DL))],
        out_specs=pl.BlockSpec((1, _DL), lambda i: (0, 0)),
        out_shape=jax.ShapeDtypeStruct((1, _DL), jnp.float32),
        scratch_shapes=[pltpu.VMEM((1, _ES), jnp.float32),
                        pltpu.VMEM((1, 1), jnp.float32)],
    )(pmask, sym, d0, d1, sumh2, w21, b21, wl, bl)


def kernel(facts, sym_emb, edge_emb, W1_0, b1_0, W2_0, b2_0,
           W1_1, b1_1, W2_1, b2_1, Wl, bl):
    pad_e = _EPAD - _E
    subj = jnp.concatenate(
        [facts[:, 0], jnp.full((pad_e,), _V, jnp.int32)]).reshape(_ER, 128)
    pred = jnp.concatenate(
        [facts[:, 1], jnp.full((pad_e,), _V, jnp.int32)]).reshape(_ER, 128)
    obj = jnp.concatenate(
        [facts[:, 2], jnp.full((pad_e,), _V, jnp.int32)]).reshape(_ER, 128)

    zpad = jnp.zeros((_VPAD - _V, _ES), jnp.float32)
    sym_p = jnp.concatenate([sym_emb, zpad], axis=0)
    edge_p = jnp.concatenate([edge_emb, zpad], axis=0)

    b1_0r = b1_0.reshape(1, 64)
    b2_0r = b2_0.reshape(1, _ES)
    b1_1r = b1_1.reshape(1, 64)
    b2_1r = b2_1.reshape(1, _ES)
    blr = bl.reshape(1, _DL)

    gs1, ge, go1 = _gather3(sym_p, edge_p, sym_p, subj, pred, obj)
    pmask = _present(subj, obj)
    msg1 = _mlp1(gs1, ge, go1, W1_0[:_ES], W1_0[_ES:2 * _ES], W1_0[2 * _ES:],
                 b1_0r, W2_0, b2_0r)
    delta = _scatter(msg1, obj)
    nodes2 = _prep2(sym_p, delta[0], delta[1])
    gs2, go2 = _gather2(nodes2, nodes2, subj, obj)
    sumh2 = _mlp2(gs2, ge, go2, W1_1[:_ES], W1_1[_ES:2 * _ES], W1_1[2 * _ES:],
                  b1_1r)
    return _final(pmask, sym_p, delta[0], delta[1], sumh2, W2_1, b2_1r,
                  Wl, blr)
